# baseline (speedup 1.0000x reference)
import functools

import jax
import jax.numpy as jnp
from jax import lax
from jax.experimental import pallas as pl
from jax.experimental.pallas import tpu as pltpu
from jax.experimental.pallas import tpu_sc as plsc

N = 10000
E = 320000
IN_DIM = 128
HIDDEN = 32

NC = 2
NS = 16
NW = NC * NS
CHUNK = 128
NBUF = 8
K = 80
T = K // NBUF
E_PAD = NW * K * CHUNK
N_PAD = 10112
ROWS_W = N_PAD // NS
DEG_W = 8

_mesh = plsc.VectorSubcoreMesh(
    core_axis_name="c", subcore_axis_name="s", num_cores=NC, num_subcores=NS
)


@functools.partial(
    pl.kernel,
    out_type=jax.ShapeDtypeStruct((NC, N_PAD, DEG_W), jnp.float32),
    mesh=_mesh,
    scratch_types=[
        pltpu.VMEM((K, CHUNK), jnp.int32),
        pltpu.VMEM((CHUNK, DEG_W), jnp.float32),
        pltpu.SemaphoreType.DMA,
        pltpu.VMEM_SHARED((N_PAD, DEG_W), jnp.float32),
    ],
    compiler_params=pltpu.CompilerParams(use_tc_tiling_on_sc=False),
)
def _deg_kernel(dst_hbm, ones_hbm, zeros_hbm, out_hbm, dst_v, ones_v, dsem, deg_sh):
    c = lax.axis_index("c")
    s = lax.axis_index("s")
    wid = c * NS + s
    pltpu.sync_copy(zeros_hbm, deg_sh.at[pl.ds(s * ROWS_W, ROWS_W)])
    pltpu.sync_copy(ones_hbm, ones_v)
    pltpu.sync_copy(dst_hbm.at[wid], dst_v)
    plsc.subcore_barrier()

    @pl.loop(0, T)
    def _(t):
        base = t * NBUF
        for i in range(NBUF):
            pltpu.async_copy(ones_v, deg_sh.at[dst_v.at[base + i]], dsem, add=True)
        for i in range(NBUF):
            pltpu.make_async_copy(ones_v, deg_sh.at[dst_v.at[base + i]], dsem).wait()

    plsc.subcore_barrier()
    pltpu.sync_copy(
        deg_sh.at[pl.ds(s * ROWS_W, ROWS_W)],
        out_hbm.at[c, pl.ds(s * ROWS_W, ROWS_W)],
    )


@functools.partial(
    pl.kernel,
    out_type=jax.ShapeDtypeStruct((NC, N_PAD, HIDDEN), jnp.float32),
    mesh=_mesh,
    scratch_types=[
        pltpu.VMEM((K, CHUNK), jnp.int32),
        pltpu.VMEM((K, CHUNK), jnp.int32),
        pltpu.VMEM((NBUF, CHUNK, HIDDEN), jnp.float32),
        pltpu.SemaphoreType.DMA((NBUF,)),
        pltpu.SemaphoreType.DMA((NBUF,)),
        pltpu.VMEM_SHARED((N_PAD, HIDDEN), jnp.float32),
    ],
    compiler_params=pltpu.CompilerParams(use_tc_tiling_on_sc=False),
)
def _agg_kernel(g_hbm, src_hbm, dst_hbm, zeros_hbm, out_hbm,
                src_v, dst_v, bufs, gsem, ssem, acc_sh):
    c = lax.axis_index("c")
    s = lax.axis_index("s")
    wid = c * NS + s
    pltpu.sync_copy(zeros_hbm, acc_sh.at[pl.ds(s * ROWS_W, ROWS_W)])
    pltpu.sync_copy(src_hbm.at[wid], src_v)
    pltpu.sync_copy(dst_hbm.at[wid], dst_v)
    plsc.subcore_barrier()

    for i in range(NBUF):
        pltpu.async_copy(g_hbm.at[src_v.at[i]], bufs.at[i], gsem.at[i])

    @pl.loop(0, T)
    def _(t):
        base = t * NBUF
        for i in range(NBUF):
            j = base + i
            pltpu.make_async_copy(g_hbm.at[src_v.at[j]], bufs.at[i], gsem.at[i]).wait()
            pltpu.async_copy(bufs.at[i], acc_sh.at[dst_v.at[j]], ssem.at[i], add=True)
        for i in range(NBUF):
            j = base + i
            pltpu.make_async_copy(bufs.at[i], acc_sh.at[dst_v.at[j]], ssem.at[i]).wait()

            @pl.when(t < T - 1)
            def _():
                pltpu.async_copy(
                    g_hbm.at[src_v.at[j + NBUF]], bufs.at[i], gsem.at[i])

    plsc.subcore_barrier()
    pltpu.sync_copy(
        acc_sh.at[pl.ds(s * ROWS_W, ROWS_W)],
        out_hbm.at[c, pl.ds(s * ROWS_W, ROWS_W)],
    )


_R = 2000


def _tc1_body(d0, d1, x, w1, dinv_out, g1_out):
    deg = d0[...][:, 0:1] + d1[...][:, 0:1] + 1.0
    dinv = lax.rsqrt(deg)
    h = jnp.dot(x[...], w1[...], preferred_element_type=jnp.float32)
    dinv_out[...] = dinv
    g1_out[...] = h * dinv


def _tc2_body(s0, s1, g1, dinv, b1, w2, g2_out):
    t = (s0[...] + s1[...] + g1[...]) * dinv[...] + b1[...]
    h = jnp.maximum(t, 0.0)
    g2_out[...] = jnp.dot(h, w2[...], preferred_element_type=jnp.float32) * dinv[...]


def _tc3_body(s0, s1, g2, dinv, b2, wl, bl, out):
    t = (s0[...] + s1[...] + g2[...]) * dinv[...] + b2[...]
    h = jnp.maximum(t, 0.0)
    out[...] = jnp.dot(h, wl[...], preferred_element_type=jnp.float32) + bl[...]


def _row_spec(w):
    return pl.BlockSpec((_R, w), lambda i: (i, 0))


def _full_spec(shape):
    return pl.BlockSpec(shape, lambda i: (0,) * len(shape))


_tc1 = pl.pallas_call(
    _tc1_body,
    grid=(N // _R,),
    in_specs=[
        _row_spec(DEG_W),
        _row_spec(DEG_W),
        _row_spec(IN_DIM),
        _full_spec((IN_DIM, HIDDEN)),
    ],
    out_specs=[_row_spec(1), _row_spec(HIDDEN)],
    out_shape=[
        jax.ShapeDtypeStruct((N, 1), jnp.float32),
        jax.ShapeDtypeStruct((N, HIDDEN), jnp.float32),
    ],
)

_tc2 = pl.pallas_call(
    _tc2_body,
    grid=(N // _R,),
    in_specs=[
        _row_spec(HIDDEN),
        _row_spec(HIDDEN),
        _row_spec(HIDDEN),
        _row_spec(1),
        _full_spec((1, HIDDEN)),
        _full_spec((HIDDEN, HIDDEN)),
    ],
    out_specs=_row_spec(HIDDEN),
    out_shape=jax.ShapeDtypeStruct((N, HIDDEN), jnp.float32),
)

_tc3 = pl.pallas_call(
    _tc3_body,
    grid=(N // _R,),
    in_specs=[
        _row_spec(HIDDEN),
        _row_spec(HIDDEN),
        _row_spec(HIDDEN),
        _row_spec(1),
        _full_spec((1, HIDDEN)),
        _full_spec((HIDDEN, 1)),
        _full_spec((1, 1)),
    ],
    out_specs=_row_spec(1),
    out_shape=jax.ShapeDtypeStruct((N, 1), jnp.float32),
)


def kernel(x, edge_index, W1, b1, W2, b2, Wl, bl):
    pad = E_PAD - E
    pad_dst = N + (jnp.arange(pad, dtype=jnp.int32) % (N_PAD - N))
    src3 = jnp.concatenate(
        [edge_index[0], jnp.zeros((pad,), jnp.int32)]).reshape(NW, K, CHUNK)
    dst3 = jnp.concatenate(
        [edge_index[1], pad_dst]).reshape(NW, K, CHUNK)
    ones_rows = jnp.ones((CHUNK, DEG_W), jnp.float32)
    zeros_deg = jnp.zeros((ROWS_W, DEG_W), jnp.float32)
    zeros_agg = jnp.zeros((ROWS_W, HIDDEN), jnp.float32)

    deg_parts = _deg_kernel(dst3, ones_rows, zeros_deg)
    dinv, g1 = _tc1(deg_parts[0, :N], deg_parts[1, :N], x, W1)

    s1 = _agg_kernel(g1, src3, dst3, zeros_agg)
    g2 = _tc2(s1[0, :N], s1[1, :N], g1, dinv, b1.reshape(1, HIDDEN), W2)

    s2 = _agg_kernel(g2, src3, dst3, zeros_agg)
    out = _tc3(s2[0, :N], s2[1, :N], g2, dinv, b2.reshape(1, HIDDEN),
               Wl, bl.reshape(1, 1))
    return out[:, 0]


# speedup vs baseline: 1.8463x; 1.8463x over previous
import functools

import jax
import jax.numpy as jnp
from jax import lax
from jax.experimental import pallas as pl
from jax.experimental.pallas import tpu as pltpu
from jax.experimental.pallas import tpu_sc as plsc

N = 10000
E = 320000
IN_DIM = 128
HIDDEN = 32

NC = 2
NS = 16
NW = NC * NS
CHUNK = 128
NBUF = 8
GROWS_W = N // NS
K = 80
T = K // NBUF
E_PAD = NW * K * CHUNK
N_PAD = 10112
ROWS_W = N_PAD // NS
DEG_W = 8

_mesh = plsc.VectorSubcoreMesh(
    core_axis_name="c", subcore_axis_name="s", num_cores=NC, num_subcores=NS
)


@functools.partial(
    pl.kernel,
    out_type=jax.ShapeDtypeStruct((NC, N_PAD, DEG_W), jnp.float32),
    mesh=_mesh,
    scratch_types=[
        pltpu.VMEM((K, CHUNK), jnp.int32),
        pltpu.VMEM((CHUNK, DEG_W), jnp.float32),
        pltpu.SemaphoreType.DMA,
        pltpu.VMEM_SHARED((N_PAD, DEG_W), jnp.float32),
    ],
    compiler_params=pltpu.CompilerParams(use_tc_tiling_on_sc=False),
)
def _deg_kernel(dst_hbm, ones_hbm, zeros_hbm, out_hbm, dst_v, ones_v, dsem, deg_sh):
    c = lax.axis_index("c")
    s = lax.axis_index("s")
    wid = c * NS + s
    pltpu.sync_copy(zeros_hbm, deg_sh.at[pl.ds(s * ROWS_W, ROWS_W)])
    pltpu.sync_copy(ones_hbm, ones_v)
    pltpu.sync_copy(dst_hbm.at[wid], dst_v)
    plsc.subcore_barrier()

    @pl.loop(0, T)
    def _(t):
        base = t * NBUF
        for i in range(NBUF):
            pltpu.async_copy(ones_v, deg_sh.at[dst_v.at[base + i]], dsem, add=True)
        for i in range(NBUF):
            pltpu.make_async_copy(ones_v, deg_sh.at[dst_v.at[base + i]], dsem).wait()

    plsc.subcore_barrier()
    pltpu.sync_copy(
        deg_sh.at[pl.ds(s * ROWS_W, ROWS_W)],
        out_hbm.at[c, pl.ds(s * ROWS_W, ROWS_W)],
    )


@functools.partial(
    pl.kernel,
    out_type=jax.ShapeDtypeStruct((NC, N_PAD, HIDDEN), jnp.float32),
    mesh=_mesh,
    scratch_types=[
        pltpu.VMEM((K, CHUNK), jnp.int32),
        pltpu.VMEM((K, CHUNK), jnp.int32),
        pltpu.VMEM((NBUF, CHUNK, HIDDEN), jnp.float32),
        pltpu.SemaphoreType.DMA((NBUF,)),
        pltpu.SemaphoreType.DMA((NBUF,)),
        pltpu.VMEM_SHARED((N_PAD, HIDDEN), jnp.float32),
        pltpu.VMEM_SHARED((N, HIDDEN), jnp.float32),
    ],
    compiler_params=pltpu.CompilerParams(use_tc_tiling_on_sc=False),
)
def _agg_kernel(g_hbm, src_hbm, dst_hbm, zeros_hbm, out_hbm,
                src_v, dst_v, bufs, gsem, ssem, acc_sh, g_sh):
    c = lax.axis_index("c")
    s = lax.axis_index("s")
    wid = c * NS + s
    pltpu.sync_copy(zeros_hbm, acc_sh.at[pl.ds(s * ROWS_W, ROWS_W)])
    pltpu.sync_copy(g_hbm.at[pl.ds(s * GROWS_W, GROWS_W)],
                    g_sh.at[pl.ds(s * GROWS_W, GROWS_W)])
    pltpu.sync_copy(src_hbm.at[wid], src_v)
    pltpu.sync_copy(dst_hbm.at[wid], dst_v)
    plsc.subcore_barrier()

    for i in range(NBUF):
        pltpu.async_copy(g_sh.at[src_v.at[i]], bufs.at[i], gsem.at[i])

    @pl.loop(0, T)
    def _(t):
        base = t * NBUF
        for i in range(NBUF):
            j = base + i
            pltpu.make_async_copy(g_sh.at[src_v.at[j]], bufs.at[i], gsem.at[i]).wait()
            pltpu.async_copy(bufs.at[i], acc_sh.at[dst_v.at[j]], ssem.at[i], add=True)
        for i in range(NBUF):
            j = base + i
            pltpu.make_async_copy(bufs.at[i], acc_sh.at[dst_v.at[j]], ssem.at[i]).wait()

            @pl.when(t < T - 1)
            def _():
                pltpu.async_copy(
                    g_sh.at[src_v.at[j + NBUF]], bufs.at[i], gsem.at[i])

    plsc.subcore_barrier()
    pltpu.sync_copy(
        acc_sh.at[pl.ds(s * ROWS_W, ROWS_W)],
        out_hbm.at[c, pl.ds(s * ROWS_W, ROWS_W)],
    )


_R = 2000


def _tc1_body(d0, d1, x, w1, dinv_out, g1_out):
    deg = d0[...][:, 0:1] + d1[...][:, 0:1] + 1.0
    dinv = lax.rsqrt(deg)
    h = jnp.dot(x[...], w1[...], preferred_element_type=jnp.float32)
    dinv_out[...] = dinv
    g1_out[...] = h * dinv


def _tc2_body(s0, s1, g1, dinv, b1, w2, g2_out):
    t = (s0[...] + s1[...] + g1[...]) * dinv[...] + b1[...]
    h = jnp.maximum(t, 0.0)
    g2_out[...] = jnp.dot(h, w2[...], preferred_element_type=jnp.float32) * dinv[...]


def _tc3_body(s0, s1, g2, dinv, b2, wl, bl, out):
    t = (s0[...] + s1[...] + g2[...]) * dinv[...] + b2[...]
    h = jnp.maximum(t, 0.0)
    out[...] = jnp.dot(h, wl[...], preferred_element_type=jnp.float32) + bl[...]


def _row_spec(w):
    return pl.BlockSpec((_R, w), lambda i: (i, 0))


def _full_spec(shape):
    return pl.BlockSpec(shape, lambda i: (0,) * len(shape))


_tc1 = pl.pallas_call(
    _tc1_body,
    grid=(N // _R,),
    in_specs=[
        _row_spec(DEG_W),
        _row_spec(DEG_W),
        _row_spec(IN_DIM),
        _full_spec((IN_DIM, HIDDEN)),
    ],
    out_specs=[_row_spec(1), _row_spec(HIDDEN)],
    out_shape=[
        jax.ShapeDtypeStruct((N, 1), jnp.float32),
        jax.ShapeDtypeStruct((N, HIDDEN), jnp.float32),
    ],
)

_tc2 = pl.pallas_call(
    _tc2_body,
    grid=(N // _R,),
    in_specs=[
        _row_spec(HIDDEN),
        _row_spec(HIDDEN),
        _row_spec(HIDDEN),
        _row_spec(1),
        _full_spec((1, HIDDEN)),
        _full_spec((HIDDEN, HIDDEN)),
    ],
    out_specs=_row_spec(HIDDEN),
    out_shape=jax.ShapeDtypeStruct((N, HIDDEN), jnp.float32),
)

_tc3 = pl.pallas_call(
    _tc3_body,
    grid=(N // _R,),
    in_specs=[
        _row_spec(HIDDEN),
        _row_spec(HIDDEN),
        _row_spec(HIDDEN),
        _row_spec(1),
        _full_spec((1, HIDDEN)),
        _full_spec((HIDDEN, 1)),
        _full_spec((1, 1)),
    ],
    out_specs=_row_spec(1),
    out_shape=jax.ShapeDtypeStruct((N, 1), jnp.float32),
)


def kernel(x, edge_index, W1, b1, W2, b2, Wl, bl):
    pad = E_PAD - E
    pad_dst = N + (jnp.arange(pad, dtype=jnp.int32) % (N_PAD - N))
    src3 = jnp.concatenate(
        [edge_index[0], jnp.zeros((pad,), jnp.int32)]).reshape(NW, K, CHUNK)
    dst3 = jnp.concatenate(
        [edge_index[1], pad_dst]).reshape(NW, K, CHUNK)
    ones_rows = jnp.ones((CHUNK, DEG_W), jnp.float32)
    zeros_deg = jnp.zeros((ROWS_W, DEG_W), jnp.float32)
    zeros_agg = jnp.zeros((ROWS_W, HIDDEN), jnp.float32)

    deg_parts = _deg_kernel(dst3, ones_rows, zeros_deg)
    dinv, g1 = _tc1(deg_parts[0, :N], deg_parts[1, :N], x, W1)

    s1 = _agg_kernel(g1, src3, dst3, zeros_agg)
    g2 = _tc2(s1[0, :N], s1[1, :N], g1, dinv, b1.reshape(1, HIDDEN), W2)

    s2 = _agg_kernel(g2, src3, dst3, zeros_agg)
    out = _tc3(s2[0, :N], s2[1, :N], g2, dinv, b2.reshape(1, HIDDEN),
               Wl, bl.reshape(1, 1))
    return out[:, 0]


# speedup vs baseline: 1.8807x; 1.0186x over previous
import functools

import jax
import jax.numpy as jnp
from jax import lax
from jax.experimental import pallas as pl
from jax.experimental.pallas import tpu as pltpu
from jax.experimental.pallas import tpu_sc as plsc

N = 10000
E = 320000
IN_DIM = 128
HIDDEN = 32

NC = 2
NS = 16
NW = NC * NS
CHUNK = 125
NBUF = 8
GROWS_W = N // NS
K = 80
T = K // NBUF
N_PAD = 10112
ROWS_W = N_PAD // NS
DEG_W = 8

_mesh = plsc.VectorSubcoreMesh(
    core_axis_name="c", subcore_axis_name="s", num_cores=NC, num_subcores=NS
)


@functools.partial(
    pl.kernel,
    out_type=jax.ShapeDtypeStruct((NC, N_PAD, DEG_W), jnp.float32),
    mesh=_mesh,
    scratch_types=[
        pltpu.VMEM((K, CHUNK), jnp.int32),
        pltpu.VMEM((CHUNK, DEG_W), jnp.float32),
        pltpu.SemaphoreType.DMA,
        pltpu.VMEM_SHARED((N_PAD, DEG_W), jnp.float32),
    ],
    compiler_params=pltpu.CompilerParams(use_tc_tiling_on_sc=False),
)
def _deg_kernel(dst_hbm, ones_hbm, zeros_hbm, out_hbm, dst_v, ones_v, dsem, deg_sh):
    c = lax.axis_index("c")
    s = lax.axis_index("s")
    wid = c * NS + s
    pltpu.sync_copy(zeros_hbm, deg_sh.at[pl.ds(s * ROWS_W, ROWS_W)])
    pltpu.sync_copy(ones_hbm, ones_v)
    pltpu.sync_copy(dst_hbm.at[wid], dst_v)
    plsc.subcore_barrier()

    @pl.loop(0, T)
    def _(t):
        base = t * NBUF
        for i in range(NBUF):
            pltpu.async_copy(ones_v, deg_sh.at[dst_v.at[base + i]], dsem, add=True)
        for i in range(NBUF):
            pltpu.make_async_copy(ones_v, deg_sh.at[dst_v.at[base + i]], dsem).wait()

    plsc.subcore_barrier()
    pltpu.sync_copy(
        deg_sh.at[pl.ds(s * ROWS_W, ROWS_W)],
        out_hbm.at[c, pl.ds(s * ROWS_W, ROWS_W)],
    )


@functools.partial(
    pl.kernel,
    out_type=jax.ShapeDtypeStruct((NC, N_PAD, HIDDEN), jnp.float32),
    mesh=_mesh,
    scratch_types=[
        pltpu.VMEM((K, CHUNK), jnp.int32),
        pltpu.VMEM((K, CHUNK), jnp.int32),
        pltpu.VMEM((NBUF, CHUNK, HIDDEN), jnp.float32),
        pltpu.SemaphoreType.DMA((NBUF,)),
        pltpu.SemaphoreType.DMA((NBUF,)),
        pltpu.VMEM_SHARED((N_PAD, HIDDEN), jnp.float32),
        pltpu.VMEM_SHARED((N, HIDDEN), jnp.float32),
    ],
    compiler_params=pltpu.CompilerParams(use_tc_tiling_on_sc=False),
)
def _agg_kernel(g_hbm, src_hbm, dst_hbm, zeros_hbm, out_hbm,
                src_v, dst_v, bufs, gsem, ssem, acc_sh, g_sh):
    c = lax.axis_index("c")
    s = lax.axis_index("s")
    wid = c * NS + s
    pltpu.sync_copy(zeros_hbm, acc_sh.at[pl.ds(s * ROWS_W, ROWS_W)])
    pltpu.sync_copy(g_hbm.at[pl.ds(s * GROWS_W, GROWS_W)],
                    g_sh.at[pl.ds(s * GROWS_W, GROWS_W)])
    pltpu.sync_copy(src_hbm.at[wid], src_v)
    pltpu.sync_copy(dst_hbm.at[wid], dst_v)
    plsc.subcore_barrier()

    for i in range(NBUF):
        pltpu.async_copy(g_sh.at[src_v.at[i]], bufs.at[i], gsem.at[i])

    @pl.loop(0, T)
    def _(t):
        base = t * NBUF
        for i in range(NBUF):
            j = base + i
            pltpu.make_async_copy(g_sh.at[src_v.at[j]], bufs.at[i], gsem.at[i]).wait()
            pltpu.async_copy(bufs.at[i], acc_sh.at[dst_v.at[j]], ssem.at[i], add=True)
        for i in range(NBUF):
            j = base + i
            pltpu.make_async_copy(bufs.at[i], acc_sh.at[dst_v.at[j]], ssem.at[i]).wait()

            @pl.when(t < T - 1)
            def _():
                pltpu.async_copy(
                    g_sh.at[src_v.at[j + NBUF]], bufs.at[i], gsem.at[i])

    plsc.subcore_barrier()
    pltpu.sync_copy(
        acc_sh.at[pl.ds(s * ROWS_W, ROWS_W)],
        out_hbm.at[c, pl.ds(s * ROWS_W, ROWS_W)],
    )


_R = 2000


def _tc0_body(x, w1, h_out):
    h_out[...] = jnp.dot(x[...], w1[...], preferred_element_type=jnp.float32)


def _tc1_body(d0, d1, h, dinv_out, g1_out):
    deg = d0[...][:, 0:1] + d1[...][:, 0:1] + 1.0
    dinv = lax.rsqrt(deg)
    dinv_out[...] = dinv
    g1_out[...] = h[...] * dinv


def _tc2_body(s0, s1, g1, dinv, b1, w2, g2_out):
    t = (s0[...] + s1[...] + g1[...]) * dinv[...] + b1[...]
    h = jnp.maximum(t, 0.0)
    g2_out[...] = jnp.dot(h, w2[...], preferred_element_type=jnp.float32) * dinv[...]


def _tc3_body(s0, s1, g2, dinv, b2, wl, bl, out):
    t = (s0[...] + s1[...] + g2[...]) * dinv[...] + b2[...]
    h = jnp.maximum(t, 0.0)
    out[...] = jnp.dot(h, wl[...], preferred_element_type=jnp.float32) + bl[...]


def _row_spec(w):
    return pl.BlockSpec((_R, w), lambda i: (i, 0))


def _full_spec(shape):
    return pl.BlockSpec(shape, lambda i: (0,) * len(shape))


_tc0 = pl.pallas_call(
    _tc0_body,
    grid=(N // _R,),
    in_specs=[
        _row_spec(IN_DIM),
        _full_spec((IN_DIM, HIDDEN)),
    ],
    out_specs=_row_spec(HIDDEN),
    out_shape=jax.ShapeDtypeStruct((N, HIDDEN), jnp.float32),
)

_tc1 = pl.pallas_call(
    _tc1_body,
    grid=(N // _R,),
    in_specs=[
        _row_spec(DEG_W),
        _row_spec(DEG_W),
        _row_spec(HIDDEN),
    ],
    out_specs=[_row_spec(1), _row_spec(HIDDEN)],
    out_shape=[
        jax.ShapeDtypeStruct((N, 1), jnp.float32),
        jax.ShapeDtypeStruct((N, HIDDEN), jnp.float32),
    ],
)

_tc2 = pl.pallas_call(
    _tc2_body,
    grid=(N // _R,),
    in_specs=[
        _row_spec(HIDDEN),
        _row_spec(HIDDEN),
        _row_spec(HIDDEN),
        _row_spec(1),
        _full_spec((1, HIDDEN)),
        _full_spec((HIDDEN, HIDDEN)),
    ],
    out_specs=_row_spec(HIDDEN),
    out_shape=jax.ShapeDtypeStruct((N, HIDDEN), jnp.float32),
)

_tc3 = pl.pallas_call(
    _tc3_body,
    grid=(N // _R,),
    in_specs=[
        _row_spec(HIDDEN),
        _row_spec(HIDDEN),
        _row_spec(HIDDEN),
        _row_spec(1),
        _full_spec((1, HIDDEN)),
        _full_spec((HIDDEN, 1)),
        _full_spec((1, 1)),
    ],
    out_specs=_row_spec(1),
    out_shape=jax.ShapeDtypeStruct((N, 1), jnp.float32),
)


def kernel(x, edge_index, W1, b1, W2, b2, Wl, bl):
    src3 = edge_index[0].reshape(NW, K, CHUNK)
    dst3 = edge_index[1].reshape(NW, K, CHUNK)
    ones_rows = jnp.ones((CHUNK, DEG_W), jnp.float32)
    zeros_deg = jnp.zeros((ROWS_W, DEG_W), jnp.float32)
    zeros_agg = jnp.zeros((ROWS_W, HIDDEN), jnp.float32)

    deg_parts = _deg_kernel(dst3, ones_rows, zeros_deg)
    h1 = _tc0(x, W1)
    dinv, g1 = _tc1(deg_parts[0, :N], deg_parts[1, :N], h1)

    s1 = _agg_kernel(g1, src3, dst3, zeros_agg)
    g2 = _tc2(s1[0, :N], s1[1, :N], g1, dinv, b1.reshape(1, HIDDEN), W2)

    s2 = _agg_kernel(g2, src3, dst3, zeros_agg)
    out = _tc3(s2[0, :N], s2[1, :N], g2, dinv, b2.reshape(1, HIDDEN),
               Wl, bl.reshape(1, 1))
    return out[:, 0]


# speedup vs baseline: 2.0462x; 1.0880x over previous
import functools

import jax
import jax.numpy as jnp
from jax import lax
from jax.experimental import pallas as pl
from jax.experimental.pallas import tpu as pltpu
from jax.experimental.pallas import tpu_sc as plsc

N = 10000
E = 320000
IN_DIM = 128
HIDDEN = 32

NC = 2
NS = 16
NW = NC * NS
CHUNK = 125
NBUF = 8
GROWS_W = N // NS
K = 80
T = K // NBUF
ROWS_W = N // NS
DEG_W = 8

_mesh = plsc.VectorSubcoreMesh(
    core_axis_name="c", subcore_axis_name="s", num_cores=NC, num_subcores=NS
)


@functools.partial(
    pl.kernel,
    out_type=jax.ShapeDtypeStruct((NC, N, DEG_W), jnp.float32),
    mesh=_mesh,
    scratch_types=[
        pltpu.VMEM((K, CHUNK), jnp.int32),
        pltpu.VMEM((CHUNK, DEG_W), jnp.float32),
        pltpu.SemaphoreType.DMA,
        pltpu.VMEM_SHARED((N, DEG_W), jnp.float32),
    ],
    compiler_params=pltpu.CompilerParams(use_tc_tiling_on_sc=False),
)
def _deg_kernel(dst_hbm, ones_hbm, zeros_hbm, out_hbm, dst_v, ones_v, dsem, deg_sh):
    c = lax.axis_index("c")
    s = lax.axis_index("s")
    wid = c * NS + s
    pltpu.sync_copy(zeros_hbm, deg_sh.at[pl.ds(s * ROWS_W, ROWS_W)])
    pltpu.sync_copy(ones_hbm, ones_v)
    pltpu.sync_copy(dst_hbm.at[wid], dst_v)
    plsc.subcore_barrier()

    @pl.loop(0, T)
    def _(t):
        base = t * NBUF
        for i in range(NBUF):
            pltpu.async_copy(ones_v, deg_sh.at[dst_v.at[base + i]], dsem, add=True)
        for i in range(NBUF):
            pltpu.make_async_copy(ones_v, deg_sh.at[dst_v.at[base + i]], dsem).wait()

    plsc.subcore_barrier()
    pltpu.sync_copy(
        deg_sh.at[pl.ds(s * ROWS_W, ROWS_W)],
        out_hbm.at[c, pl.ds(s * ROWS_W, ROWS_W)],
    )


@functools.partial(
    pl.kernel,
    out_type=jax.ShapeDtypeStruct((NC, N, HIDDEN), jnp.float32),
    mesh=_mesh,
    scratch_types=[
        pltpu.VMEM((K, CHUNK), jnp.int32),
        pltpu.VMEM((K, CHUNK), jnp.int32),
        pltpu.VMEM((NBUF, CHUNK, HIDDEN), jnp.float32),
        pltpu.SemaphoreType.DMA((NBUF,)),
        pltpu.SemaphoreType.DMA((NBUF,)),
        pltpu.VMEM_SHARED((N, HIDDEN), jnp.float32),
        pltpu.VMEM_SHARED((N, HIDDEN), jnp.float32),
    ],
    compiler_params=pltpu.CompilerParams(use_tc_tiling_on_sc=False),
)
def _agg_kernel(g_hbm, src_hbm, dst_hbm, zeros_hbm, out_hbm,
                src_v, dst_v, bufs, gsem, ssem, acc_sh, g_sh):
    c = lax.axis_index("c")
    s = lax.axis_index("s")
    wid = c * NS + s
    pltpu.sync_copy(zeros_hbm, acc_sh.at[pl.ds(s * ROWS_W, ROWS_W)])
    pltpu.sync_copy(g_hbm.at[pl.ds(s * GROWS_W, GROWS_W)],
                    g_sh.at[pl.ds(s * GROWS_W, GROWS_W)])
    pltpu.sync_copy(src_hbm.at[wid], src_v)
    pltpu.sync_copy(dst_hbm.at[wid], dst_v)
    plsc.subcore_barrier()

    for i in range(NBUF):
        pltpu.async_copy(g_sh.at[src_v.at[i]], bufs.at[i], gsem.at[i])

    @pl.loop(0, T)
    def _(t):
        base = t * NBUF
        for i in range(NBUF):
            j = base + i
            pltpu.make_async_copy(g_sh.at[src_v.at[j]], bufs.at[i], gsem.at[i]).wait()
            pltpu.async_copy(bufs.at[i], acc_sh.at[dst_v.at[j]], ssem.at[i], add=True)
        for i in range(NBUF):
            j = base + i
            pltpu.make_async_copy(bufs.at[i], acc_sh.at[dst_v.at[j]], ssem.at[i]).wait()

            @pl.when(t < T - 1)
            def _():
                pltpu.async_copy(
                    g_sh.at[src_v.at[j + NBUF]], bufs.at[i], gsem.at[i])

    plsc.subcore_barrier()
    pltpu.sync_copy(
        acc_sh.at[pl.ds(s * ROWS_W, ROWS_W)],
        out_hbm.at[c, pl.ds(s * ROWS_W, ROWS_W)],
    )


_R = 2000


def _tc0_body(x, w1, h_out):
    h_out[...] = jnp.dot(x[...], w1[...], preferred_element_type=jnp.float32)


def _tc1_body(d, h, dinv_out, g1_out):
    dd = d[...]
    deg = dd[0][:, 0:1] + dd[1][:, 0:1] + 1.0
    dinv = lax.rsqrt(deg)
    dinv_out[...] = dinv
    g1_out[...] = h[...] * dinv


def _tc2_body(sp, g1, dinv, b1, w2, g2_out):
    ss = sp[...]
    t = (ss[0] + ss[1] + g1[...]) * dinv[...] + b1[...]
    h = jnp.maximum(t, 0.0)
    g2_out[...] = jnp.dot(h, w2[...], preferred_element_type=jnp.float32) * dinv[...]


def _tc3_body(sp, g2, dinv, b2, wl, bl, out):
    ss = sp[...]
    t = (ss[0] + ss[1] + g2[...]) * dinv[...] + b2[...]
    h = jnp.maximum(t, 0.0)
    out[...] = jnp.dot(h, wl[...], preferred_element_type=jnp.float32) + bl[...]


def _row_spec(w):
    return pl.BlockSpec((_R, w), lambda i: (i, 0))


def _parts_spec(w):
    return pl.BlockSpec((NC, _R, w), lambda i: (0, i, 0))


def _full_spec(shape):
    return pl.BlockSpec(shape, lambda i: (0,) * len(shape))


_tc0 = pl.pallas_call(
    _tc0_body,
    grid=(N // _R,),
    in_specs=[
        _row_spec(IN_DIM),
        _full_spec((IN_DIM, HIDDEN)),
    ],
    out_specs=_row_spec(HIDDEN),
    out_shape=jax.ShapeDtypeStruct((N, HIDDEN), jnp.float32),
)

_tc1 = pl.pallas_call(
    _tc1_body,
    grid=(N // _R,),
    in_specs=[
        _parts_spec(DEG_W),
        _row_spec(HIDDEN),
    ],
    out_specs=[_row_spec(1), _row_spec(HIDDEN)],
    out_shape=[
        jax.ShapeDtypeStruct((N, 1), jnp.float32),
        jax.ShapeDtypeStruct((N, HIDDEN), jnp.float32),
    ],
)

_tc2 = pl.pallas_call(
    _tc2_body,
    grid=(N // _R,),
    in_specs=[
        _parts_spec(HIDDEN),
        _row_spec(HIDDEN),
        _row_spec(1),
        _full_spec((1, HIDDEN)),
        _full_spec((HIDDEN, HIDDEN)),
    ],
    out_specs=_row_spec(HIDDEN),
    out_shape=jax.ShapeDtypeStruct((N, HIDDEN), jnp.float32),
)

_tc3 = pl.pallas_call(
    _tc3_body,
    grid=(N // _R,),
    in_specs=[
        _parts_spec(HIDDEN),
        _row_spec(HIDDEN),
        _row_spec(1),
        _full_spec((1, HIDDEN)),
        _full_spec((HIDDEN, 1)),
        _full_spec((1, 1)),
    ],
    out_specs=_row_spec(1),
    out_shape=jax.ShapeDtypeStruct((N, 1), jnp.float32),
)


def kernel(x, edge_index, W1, b1, W2, b2, Wl, bl):
    src3 = edge_index[0].reshape(NW, K, CHUNK)
    dst3 = edge_index[1].reshape(NW, K, CHUNK)
    ones_rows = jnp.ones((CHUNK, DEG_W), jnp.float32)
    zeros_deg = jnp.zeros((ROWS_W, DEG_W), jnp.float32)
    zeros_agg = jnp.zeros((ROWS_W, HIDDEN), jnp.float32)

    deg_parts = _deg_kernel(dst3, ones_rows, zeros_deg)
    h1 = _tc0(x, W1)
    dinv, g1 = _tc1(deg_parts, h1)

    s1 = _agg_kernel(g1, src3, dst3, zeros_agg)
    g2 = _tc2(s1, g1, dinv, b1.reshape(1, HIDDEN), W2)

    s2 = _agg_kernel(g2, src3, dst3, zeros_agg)
    out = _tc3(s2, g2, dinv, b2.reshape(1, HIDDEN), Wl, bl.reshape(1, 1))
    return out[:, 0]


# speedup vs baseline: 2.0505x; 1.0021x over previous
import functools

import jax
import jax.numpy as jnp
from jax import lax
from jax.experimental import pallas as pl
from jax.experimental.pallas import tpu as pltpu
from jax.experimental.pallas import tpu_sc as plsc

N = 10000
E = 320000
IN_DIM = 128
HIDDEN = 32

NC = 2
NS = 16
NW = NC * NS
CHUNK = 125
NBUF = 8
GROWS_W = N // NS
K = 80
T = K // NBUF
ROWS_W = N // NS
DEG_W = 8

_mesh = plsc.VectorSubcoreMesh(
    core_axis_name="c", subcore_axis_name="s", num_cores=NC, num_subcores=NS
)


@functools.partial(
    pl.kernel,
    out_type=jax.ShapeDtypeStruct((NC, N, DEG_W), jnp.float32),
    mesh=_mesh,
    scratch_types=[
        pltpu.VMEM((K, CHUNK), jnp.int32),
        pltpu.VMEM((CHUNK, DEG_W), jnp.float32),
        pltpu.SemaphoreType.DMA,
        pltpu.VMEM_SHARED((N, DEG_W), jnp.float32),
    ],
    compiler_params=pltpu.CompilerParams(use_tc_tiling_on_sc=False),
)
def _deg_kernel(dst_hbm, ones_hbm, zeros_hbm, out_hbm, dst_v, ones_v, dsem, deg_sh):
    c = lax.axis_index("c")
    s = lax.axis_index("s")
    wid = c * NS + s
    pltpu.sync_copy(zeros_hbm, deg_sh.at[pl.ds(s * ROWS_W, ROWS_W)])
    pltpu.sync_copy(ones_hbm, ones_v)
    pltpu.sync_copy(dst_hbm.at[wid], dst_v)
    plsc.subcore_barrier()

    @pl.loop(0, T)
    def _(t):
        base = t * NBUF
        for i in range(NBUF):
            pltpu.async_copy(ones_v, deg_sh.at[dst_v.at[base + i]], dsem, add=True)
        for i in range(NBUF):
            pltpu.make_async_copy(ones_v, deg_sh.at[dst_v.at[base + i]], dsem).wait()

    plsc.subcore_barrier()
    pltpu.sync_copy(
        deg_sh.at[pl.ds(s * ROWS_W, ROWS_W)],
        out_hbm.at[c, pl.ds(s * ROWS_W, ROWS_W)],
    )


@functools.partial(
    pl.kernel,
    out_type=jax.ShapeDtypeStruct((NC, N, HIDDEN), jnp.float32),
    mesh=_mesh,
    scratch_types=[
        pltpu.VMEM((K, CHUNK), jnp.int32),
        pltpu.VMEM((K, CHUNK), jnp.int32),
        pltpu.VMEM((NBUF, CHUNK, HIDDEN), jnp.float32),
        pltpu.SemaphoreType.DMA((NBUF,)),
        pltpu.SemaphoreType.DMA((NBUF,)),
        pltpu.VMEM_SHARED((N, HIDDEN), jnp.float32),
        pltpu.VMEM_SHARED((N, HIDDEN), jnp.float32),
    ],
    compiler_params=pltpu.CompilerParams(use_tc_tiling_on_sc=False),
)
def _agg_kernel(g_hbm, src_hbm, dst_hbm, zeros_hbm, out_hbm,
                src_v, dst_v, bufs, gsem, ssem, acc_sh, g_sh):
    c = lax.axis_index("c")
    s = lax.axis_index("s")
    wid = c * NS + s
    pltpu.sync_copy(zeros_hbm, acc_sh.at[pl.ds(s * ROWS_W, ROWS_W)])
    pltpu.sync_copy(g_hbm.at[pl.ds(s * GROWS_W, GROWS_W)],
                    g_sh.at[pl.ds(s * GROWS_W, GROWS_W)])
    pltpu.sync_copy(src_hbm.at[wid], src_v)
    pltpu.sync_copy(dst_hbm.at[wid], dst_v)
    plsc.subcore_barrier()

    for i in range(NBUF):
        pltpu.async_copy(g_sh.at[src_v.at[i]], bufs.at[i], gsem.at[i])

    @pl.loop(0, T)
    def _(t):
        base = t * NBUF
        for i in range(NBUF):
            j = base + i
            pltpu.make_async_copy(g_sh.at[src_v.at[j]], bufs.at[i], gsem.at[i]).wait()
            pltpu.async_copy(bufs.at[i], acc_sh.at[dst_v.at[j]], ssem.at[i], add=True)
        for i in range(NBUF):
            j = base + i
            pltpu.make_async_copy(bufs.at[i], acc_sh.at[dst_v.at[j]], ssem.at[i]).wait()

            @pl.when(t < T - 1)
            def _():
                pltpu.async_copy(
                    g_sh.at[src_v.at[j + NBUF]], bufs.at[i], gsem.at[i])

    plsc.subcore_barrier()
    pltpu.sync_copy(
        acc_sh.at[pl.ds(s * ROWS_W, ROWS_W)],
        out_hbm.at[c, pl.ds(s * ROWS_W, ROWS_W)],
    )


_R = 2000


def _tc0_body(x, w1, h_out):
    h_out[...] = jnp.dot(x[...], w1[...], preferred_element_type=jnp.float32)


def _tc1_body(d, h, dinv_out, g1_out):
    dd = d[...]
    deg = dd[0][:, 0:1] + dd[1][:, 0:1] + 1.0
    dinv = lax.rsqrt(deg)
    dinv_out[...] = dinv
    g1_out[...] = h[...] * dinv


def _tc2_body(sp, g1, dinv, b1, w2, g2_out):
    ss = sp[...]
    t = (ss[0] + ss[1] + g1[...]) * dinv[...] + b1[...]
    h = jnp.maximum(t, 0.0)
    g2_out[...] = jnp.dot(h, w2[...], preferred_element_type=jnp.float32) * dinv[...]


def _tc3_body(sp, g2, dinv, b2, wl, bl, out):
    ss = sp[...]
    t = (ss[0] + ss[1] + g2[...]) * dinv[...] + b2[...]
    h = jnp.maximum(t, 0.0)
    out[...] = jnp.dot(h, wl[...], preferred_element_type=jnp.float32) + bl[...]


def _row_spec(w):
    return pl.BlockSpec((_R, w), lambda i: (i, 0))


def _parts_spec(w):
    return pl.BlockSpec((NC, _R, w), lambda i: (0, i, 0))


def _full_spec(shape):
    return pl.BlockSpec(shape, lambda i: (0,) * len(shape))


_tc0 = pl.pallas_call(
    _tc0_body,
    grid=(N // _R,),
    in_specs=[
        _row_spec(IN_DIM),
        _full_spec((IN_DIM, HIDDEN)),
    ],
    out_specs=_row_spec(HIDDEN),
    out_shape=jax.ShapeDtypeStruct((N, HIDDEN), jnp.float32),
)

_tc1 = pl.pallas_call(
    _tc1_body,
    out_shape=[
        jax.ShapeDtypeStruct((N, 1), jnp.float32),
        jax.ShapeDtypeStruct((N, HIDDEN), jnp.float32),
    ],
)

_tc2 = pl.pallas_call(
    _tc2_body,
    out_shape=jax.ShapeDtypeStruct((N, HIDDEN), jnp.float32),
)

_tc3 = pl.pallas_call(
    _tc3_body,
    out_shape=jax.ShapeDtypeStruct((N, 1), jnp.float32),
)


def kernel(x, edge_index, W1, b1, W2, b2, Wl, bl):
    src3 = edge_index[0].reshape(NW, K, CHUNK)
    dst3 = edge_index[1].reshape(NW, K, CHUNK)
    ones_rows = jnp.ones((CHUNK, DEG_W), jnp.float32)
    zeros_deg = jnp.zeros((ROWS_W, DEG_W), jnp.float32)
    zeros_agg = jnp.zeros((ROWS_W, HIDDEN), jnp.float32)

    deg_parts = _deg_kernel(dst3, ones_rows, zeros_deg)
    h1 = _tc0(x, W1)
    dinv, g1 = _tc1(deg_parts, h1)

    s1 = _agg_kernel(g1, src3, dst3, zeros_agg)
    g2 = _tc2(s1, g1, dinv, b1.reshape(1, HIDDEN), W2)

    s2 = _agg_kernel(g2, src3, dst3, zeros_agg)
    out = _tc3(s2, g2, dinv, b2.reshape(1, HIDDEN), Wl, bl.reshape(1, 1))
    return out[:, 0]


# speedup vs baseline: 2.1019x; 1.0251x over previous
import functools

import jax
import jax.numpy as jnp
from jax import lax
from jax.experimental import pallas as pl
from jax.experimental.pallas import tpu as pltpu
from jax.experimental.pallas import tpu_sc as plsc

N = 10000
E = 320000
IN_DIM = 128
HIDDEN = 32

NC = 2
NS = 16
NW = NC * NS
CHUNK = 80
NBUF = 5
GROWS_W = N // NS
EW = E // NW
K = EW // CHUNK
T = K // NBUF
ROWS_W = N // NS
DEG_W = 8

_mesh = plsc.VectorSubcoreMesh(
    core_axis_name="c", subcore_axis_name="s", num_cores=NC, num_subcores=NS
)


@functools.partial(
    pl.kernel,
    out_type=jax.ShapeDtypeStruct((NC, N, DEG_W), jnp.float32),
    mesh=_mesh,
    scratch_types=[
        pltpu.VMEM((EW,), jnp.int32),
        pltpu.VMEM((CHUNK, DEG_W), jnp.float32),
        pltpu.SemaphoreType.DMA,
        pltpu.VMEM_SHARED((N, DEG_W), jnp.float32),
    ],
    compiler_params=pltpu.CompilerParams(use_tc_tiling_on_sc=False),
)
def _deg_kernel(edges_hbm, ones_hbm, zeros_hbm, out_hbm, dst_v, ones_v, dsem, deg_sh):
    c = lax.axis_index("c")
    s = lax.axis_index("s")
    wid = c * NS + s
    pltpu.sync_copy(zeros_hbm, deg_sh.at[pl.ds(s * ROWS_W, ROWS_W)])
    pltpu.sync_copy(ones_hbm, ones_v)
    pltpu.sync_copy(edges_hbm.at[1, pl.ds(wid * EW, EW)], dst_v)
    plsc.subcore_barrier()

    @pl.loop(0, T)
    def _(t):
        base = t * NBUF
        for i in range(NBUF):
            pltpu.async_copy(
                ones_v, deg_sh.at[dst_v.at[pl.ds((base + i) * CHUNK, CHUNK)]],
                dsem, add=True)
        for i in range(NBUF):
            pltpu.make_async_copy(
                ones_v, deg_sh.at[dst_v.at[pl.ds((base + i) * CHUNK, CHUNK)]],
                dsem).wait()

    plsc.subcore_barrier()
    pltpu.sync_copy(
        deg_sh.at[pl.ds(s * ROWS_W, ROWS_W)],
        out_hbm.at[c, pl.ds(s * ROWS_W, ROWS_W)],
    )


@functools.partial(
    pl.kernel,
    out_type=jax.ShapeDtypeStruct((NC, N, HIDDEN), jnp.float32),
    mesh=_mesh,
    scratch_types=[
        pltpu.VMEM((EW,), jnp.int32),
        pltpu.VMEM((EW,), jnp.int32),
        pltpu.VMEM((NBUF, CHUNK, HIDDEN), jnp.float32),
        pltpu.SemaphoreType.DMA((NBUF,)),
        pltpu.SemaphoreType.DMA((NBUF,)),
        pltpu.VMEM_SHARED((N, HIDDEN), jnp.float32),
        pltpu.VMEM_SHARED((N, HIDDEN), jnp.float32),
    ],
    compiler_params=pltpu.CompilerParams(use_tc_tiling_on_sc=False),
)
def _agg_kernel(g_hbm, edges_hbm, zeros_hbm, out_hbm,
                src_v, dst_v, bufs, gsem, ssem, acc_sh, g_sh):

    def src_at(j):
        return src_v.at[pl.ds(j * CHUNK, CHUNK)]

    def dst_at(j):
        return dst_v.at[pl.ds(j * CHUNK, CHUNK)]

    c = lax.axis_index("c")
    s = lax.axis_index("s")
    wid = c * NS + s
    pltpu.sync_copy(zeros_hbm, acc_sh.at[pl.ds(s * ROWS_W, ROWS_W)])
    pltpu.sync_copy(g_hbm.at[pl.ds(s * GROWS_W, GROWS_W)],
                    g_sh.at[pl.ds(s * GROWS_W, GROWS_W)])
    pltpu.sync_copy(edges_hbm.at[0, pl.ds(wid * EW, EW)], src_v)
    pltpu.sync_copy(edges_hbm.at[1, pl.ds(wid * EW, EW)], dst_v)
    plsc.subcore_barrier()

    for i in range(NBUF):
        pltpu.async_copy(g_sh.at[src_at(i)], bufs.at[i], gsem.at[i])

    @pl.loop(0, T)
    def _(t):
        base = t * NBUF
        for i in range(NBUF):
            j = base + i
            pltpu.make_async_copy(g_sh.at[src_at(j)], bufs.at[i], gsem.at[i]).wait()
            pltpu.async_copy(bufs.at[i], acc_sh.at[dst_at(j)], ssem.at[i], add=True)
        for i in range(NBUF):
            j = base + i
            pltpu.make_async_copy(bufs.at[i], acc_sh.at[dst_at(j)], ssem.at[i]).wait()

            @pl.when(t < T - 1)
            def _():
                pltpu.async_copy(g_sh.at[src_at(j + NBUF)], bufs.at[i], gsem.at[i])

    plsc.subcore_barrier()
    pltpu.sync_copy(
        acc_sh.at[pl.ds(s * ROWS_W, ROWS_W)],
        out_hbm.at[c, pl.ds(s * ROWS_W, ROWS_W)],
    )


_R = 2000


def _tc0_body(x, w1, h_out):
    h_out[...] = jnp.dot(x[...], w1[...], preferred_element_type=jnp.float32)


def _tc1_body(d, h, dinv_out, g1_out):
    dd = d[...]
    deg = dd[0][:, 0:1] + dd[1][:, 0:1] + 1.0
    dinv = lax.rsqrt(deg)
    dinv_out[...] = dinv
    g1_out[...] = h[...] * dinv


def _tc2_body(sp, g1, dinv, b1, w2, g2_out):
    ss = sp[...]
    t = (ss[0] + ss[1] + g1[...]) * dinv[...] + b1[...]
    h = jnp.maximum(t, 0.0)
    g2_out[...] = jnp.dot(h, w2[...], preferred_element_type=jnp.float32) * dinv[...]


def _tc3_body(sp, g2, dinv, b2, wl, bl, out):
    ss = sp[...]
    t = (ss[0] + ss[1] + g2[...]) * dinv[...] + b2[...]
    h = jnp.maximum(t, 0.0)
    out[...] = jnp.dot(h, wl[...], preferred_element_type=jnp.float32) + bl[...]


def _row_spec(w):
    return pl.BlockSpec((_R, w), lambda i: (i, 0))


def _parts_spec(w):
    return pl.BlockSpec((NC, _R, w), lambda i: (0, i, 0))


def _full_spec(shape):
    return pl.BlockSpec(shape, lambda i: (0,) * len(shape))


_tc0 = pl.pallas_call(
    _tc0_body,
    grid=(N // _R,),
    in_specs=[
        _row_spec(IN_DIM),
        _full_spec((IN_DIM, HIDDEN)),
    ],
    out_specs=_row_spec(HIDDEN),
    out_shape=jax.ShapeDtypeStruct((N, HIDDEN), jnp.float32),
)

_tc1 = pl.pallas_call(
    _tc1_body,
    out_shape=[
        jax.ShapeDtypeStruct((N, 1), jnp.float32),
        jax.ShapeDtypeStruct((N, HIDDEN), jnp.float32),
    ],
)

_tc2 = pl.pallas_call(
    _tc2_body,
    out_shape=jax.ShapeDtypeStruct((N, HIDDEN), jnp.float32),
)

_tc3 = pl.pallas_call(
    _tc3_body,
    out_shape=jax.ShapeDtypeStruct((N, 1), jnp.float32),
)


def kernel(x, edge_index, W1, b1, W2, b2, Wl, bl):
    ones_rows = jnp.ones((CHUNK, DEG_W), jnp.float32)
    zeros_deg = jnp.zeros((ROWS_W, DEG_W), jnp.float32)
    zeros_agg = jnp.zeros((ROWS_W, HIDDEN), jnp.float32)

    deg_parts = _deg_kernel(edge_index, ones_rows, zeros_deg)
    h1 = _tc0(x, W1)
    dinv, g1 = _tc1(deg_parts, h1)

    s1 = _agg_kernel(g1, edge_index, zeros_agg)
    g2 = _tc2(s1, g1, dinv, b1.reshape(1, HIDDEN), W2)

    s2 = _agg_kernel(g2, edge_index, zeros_agg)
    out = _tc3(s2, g2, dinv, b2.reshape(1, HIDDEN), Wl, bl.reshape(1, 1))
    return out[:, 0]


# speedup vs baseline: 2.1571x; 1.0263x over previous
import functools

import jax
import jax.numpy as jnp
from jax import lax
from jax.experimental import pallas as pl
from jax.experimental.pallas import tpu as pltpu
from jax.experimental.pallas import tpu_sc as plsc

N = 10000
E = 320000
IN_DIM = 128
HIDDEN = 32

NC = 2
NS = 16
NW = NC * NS
CHUNK = 80
NBUF = 8
GROWS_W = N // NS
EW = E // NW
K = EW // CHUNK
T = K // NBUF
ROWS_W = N // NS
DEG_W = 8
KD = K // NBUF

_mesh = plsc.VectorSubcoreMesh(
    core_axis_name="c", subcore_axis_name="s", num_cores=NC, num_subcores=NS
)


@functools.partial(
    pl.kernel,
    out_type=jax.ShapeDtypeStruct((NC, N, DEG_W), jnp.float32),
    mesh=_mesh,
    scratch_types=[
        pltpu.VMEM((EW,), jnp.int32),
        pltpu.VMEM((CHUNK, DEG_W), jnp.float32),
        pltpu.SemaphoreType.DMA,
        pltpu.VMEM_SHARED((N, DEG_W), jnp.float32),
    ],
    compiler_params=pltpu.CompilerParams(use_tc_tiling_on_sc=False),
)
def _deg_kernel(edges_hbm, ones_hbm, zeros_hbm, out_hbm, dst_v, ones_v, dsem, deg_sh):
    c = lax.axis_index("c")
    s = lax.axis_index("s")
    wid = c * NS + s
    pltpu.sync_copy(zeros_hbm, deg_sh.at[pl.ds(s * ROWS_W, ROWS_W)])
    pltpu.sync_copy(ones_hbm, ones_v)
    pltpu.sync_copy(edges_hbm.at[1, pl.ds(wid * EW, EW)], dst_v)
    plsc.subcore_barrier()

    @pl.loop(0, KD)
    def _(t):
        base = t * NBUF
        for i in range(NBUF):
            pltpu.async_copy(
                ones_v, deg_sh.at[dst_v.at[pl.ds((base + i) * CHUNK, CHUNK)]],
                dsem, add=True)
        for i in range(NBUF):
            pltpu.make_async_copy(
                ones_v, deg_sh.at[dst_v.at[pl.ds((base + i) * CHUNK, CHUNK)]],
                dsem).wait()

    for j in range(KD * NBUF, K):
        pltpu.async_copy(
            ones_v, deg_sh.at[dst_v.at[pl.ds(j * CHUNK, CHUNK)]], dsem, add=True)
    for j in range(KD * NBUF, K):
        pltpu.make_async_copy(
            ones_v, deg_sh.at[dst_v.at[pl.ds(j * CHUNK, CHUNK)]], dsem).wait()

    plsc.subcore_barrier()
    pltpu.sync_copy(
        deg_sh.at[pl.ds(s * ROWS_W, ROWS_W)],
        out_hbm.at[c, pl.ds(s * ROWS_W, ROWS_W)],
    )


@functools.partial(
    pl.kernel,
    out_type=jax.ShapeDtypeStruct((NC, N, HIDDEN), jnp.float32),
    mesh=_mesh,
    scratch_types=[
        pltpu.VMEM((EW,), jnp.int32),
        pltpu.VMEM((EW,), jnp.int32),
        pltpu.VMEM((NBUF, CHUNK, HIDDEN), jnp.float32),
        pltpu.SemaphoreType.DMA((NBUF,)),
        pltpu.SemaphoreType.DMA((NBUF,)),
        pltpu.VMEM_SHARED((N, HIDDEN), jnp.float32),
        pltpu.VMEM_SHARED((N, HIDDEN), jnp.float32),
    ],
    compiler_params=pltpu.CompilerParams(use_tc_tiling_on_sc=False),
)
def _agg_kernel(g_hbm, edges_hbm, zeros_hbm, out_hbm,
                src_v, dst_v, bufs, gsem, ssem, acc_sh, g_sh):

    def src_at(j):
        return src_v.at[pl.ds(j * CHUNK, CHUNK)]

    def dst_at(j):
        return dst_v.at[pl.ds(j * CHUNK, CHUNK)]

    c = lax.axis_index("c")
    s = lax.axis_index("s")
    wid = c * NS + s
    pltpu.sync_copy(zeros_hbm, acc_sh.at[pl.ds(s * ROWS_W, ROWS_W)])
    pltpu.sync_copy(g_hbm.at[pl.ds(s * GROWS_W, GROWS_W)],
                    g_sh.at[pl.ds(s * GROWS_W, GROWS_W)])
    pltpu.sync_copy(edges_hbm.at[0, pl.ds(wid * EW, EW)], src_v)
    pltpu.sync_copy(edges_hbm.at[1, pl.ds(wid * EW, EW)], dst_v)
    plsc.subcore_barrier()

    for i in range(NBUF):
        pltpu.async_copy(g_sh.at[src_at(i)], bufs.at[i], gsem.at[i])

    @pl.loop(0, T)
    def _(t):
        base = t * NBUF
        for i in range(NBUF):
            j = base + i
            pltpu.make_async_copy(g_sh.at[src_at(j)], bufs.at[i], gsem.at[i]).wait()
            pltpu.async_copy(bufs.at[i], acc_sh.at[dst_at(j)], ssem.at[i], add=True)
        for i in range(NBUF):
            j = base + i
            pltpu.make_async_copy(bufs.at[i], acc_sh.at[dst_at(j)], ssem.at[i]).wait()

            @pl.when(j + NBUF < K)
            def _():
                pltpu.async_copy(g_sh.at[src_at(j + NBUF)], bufs.at[i], gsem.at[i])

    for j in range(T * NBUF, K):
        i = j - T * NBUF
        pltpu.make_async_copy(g_sh.at[src_at(j)], bufs.at[i], gsem.at[i]).wait()
        pltpu.async_copy(bufs.at[i], acc_sh.at[dst_at(j)], ssem.at[i], add=True)
    for j in range(T * NBUF, K):
        i = j - T * NBUF
        pltpu.make_async_copy(bufs.at[i], acc_sh.at[dst_at(j)], ssem.at[i]).wait()

    plsc.subcore_barrier()
    pltpu.sync_copy(
        acc_sh.at[pl.ds(s * ROWS_W, ROWS_W)],
        out_hbm.at[c, pl.ds(s * ROWS_W, ROWS_W)],
    )


_R = 2000


def _tc0_body(x, w1, h_out):
    h_out[...] = jnp.dot(x[...], w1[...], preferred_element_type=jnp.float32)


def _tc1_body(d, h, dinv_out, g1_out):
    dd = d[...]
    deg = dd[0][:, 0:1] + dd[1][:, 0:1] + 1.0
    dinv = lax.rsqrt(deg)
    dinv_out[...] = dinv
    g1_out[...] = h[...] * dinv


def _tc2_body(sp, g1, dinv, b1, w2, g2_out):
    ss = sp[...]
    t = (ss[0] + ss[1] + g1[...]) * dinv[...] + b1[...]
    h = jnp.maximum(t, 0.0)
    g2_out[...] = jnp.dot(h, w2[...], preferred_element_type=jnp.float32) * dinv[...]


def _tc3_body(sp, g2, dinv, b2, wl, bl, out):
    ss = sp[...]
    t = (ss[0] + ss[1] + g2[...]) * dinv[...] + b2[...]
    h = jnp.maximum(t, 0.0)
    out[...] = jnp.dot(h, wl[...], preferred_element_type=jnp.float32) + bl[...]


def _row_spec(w):
    return pl.BlockSpec((_R, w), lambda i: (i, 0))


def _parts_spec(w):
    return pl.BlockSpec((NC, _R, w), lambda i: (0, i, 0))


def _full_spec(shape):
    return pl.BlockSpec(shape, lambda i: (0,) * len(shape))


_tc0 = pl.pallas_call(
    _tc0_body,
    grid=(N // _R,),
    in_specs=[
        _row_spec(IN_DIM),
        _full_spec((IN_DIM, HIDDEN)),
    ],
    out_specs=_row_spec(HIDDEN),
    out_shape=jax.ShapeDtypeStruct((N, HIDDEN), jnp.float32),
)

_tc1 = pl.pallas_call(
    _tc1_body,
    out_shape=[
        jax.ShapeDtypeStruct((N, 1), jnp.float32),
        jax.ShapeDtypeStruct((N, HIDDEN), jnp.float32),
    ],
)

_tc2 = pl.pallas_call(
    _tc2_body,
    out_shape=jax.ShapeDtypeStruct((N, HIDDEN), jnp.float32),
)

_tc3 = pl.pallas_call(
    _tc3_body,
    out_shape=jax.ShapeDtypeStruct((N, 1), jnp.float32),
)


def kernel(x, edge_index, W1, b1, W2, b2, Wl, bl):
    ones_rows = jnp.ones((CHUNK, DEG_W), jnp.float32)
    zeros_deg = jnp.zeros((ROWS_W, DEG_W), jnp.float32)
    zeros_agg = jnp.zeros((ROWS_W, HIDDEN), jnp.float32)

    deg_parts = _deg_kernel(edge_index, ones_rows, zeros_deg)
    h1 = _tc0(x, W1)
    dinv, g1 = _tc1(deg_parts, h1)

    s1 = _agg_kernel(g1, edge_index, zeros_agg)
    g2 = _tc2(s1, g1, dinv, b1.reshape(1, HIDDEN), W2)

    s2 = _agg_kernel(g2, edge_index, zeros_agg)
    out = _tc3(s2, g2, dinv, b2.reshape(1, HIDDEN), Wl, bl.reshape(1, 1))
    return out[:, 0]
